# Initial kernel scaffold; baseline (speedup 1.0000x reference)
import jax
import jax.numpy as jnp
from jax.experimental import pallas as pl


def kernel(tokens, table):
    raise NotImplementedError("write your pallas kernel here")


# speedup vs baseline: 5.3616x; 5.3616x over previous
import functools
import math

import jax
import jax.numpy as jnp
from jax import lax
from jax.experimental import pallas as pl
from jax.experimental.pallas import tpu as pltpu
from jax.experimental.pallas import tpu_sc as plsc


_NC = 2
_NS = 16
_NW = _NC * _NS

_CH = 128


def _scale_table(table, scale):
    v, d = table.shape
    block = 800
    assert v % block == 0

    def body(t_ref, o_ref):
        o_ref[...] = t_ref[...] * scale

    return pl.pallas_call(
        body,
        grid=(v // block,),
        in_specs=[pl.BlockSpec((block, d), lambda i: (i, 0))],
        out_specs=pl.BlockSpec((block, d), lambda i: (i, 0)),
        out_shape=jax.ShapeDtypeStruct((v, d), table.dtype),
    )(table)


def _make_sc_gather(n, v, d):
    assert n % (_NW * _CH) == 0
    chunks_per_w = n // (_NW * _CH)
    mesh = plsc.VectorSubcoreMesh(core_axis_name="c", subcore_axis_name="s")

    @functools.partial(
        pl.kernel,
        mesh=mesh,
        out_type=jax.ShapeDtypeStruct((n, d), jnp.float32),
        scratch_types=[
            pltpu.VMEM((chunks_per_w, _CH), jnp.int32),
            pltpu.VMEM((_CH, d), jnp.float32),
            pltpu.SemaphoreType.DMA,
        ],
    )
    def k(table_hbm, idx_hbm, out_hbm, idx_v, rows_v, sem):
        wid = lax.axis_index("s") * _NC + lax.axis_index("c")
        base = wid * (chunks_per_w * _CH)
        pltpu.sync_copy(idx_hbm.at[wid], idx_v)

        def body(g, _):
            pltpu.async_copy(table_hbm.at[idx_v.at[g]], rows_v, sem).wait()
            pltpu.sync_copy(rows_v, out_hbm.at[pl.ds(base + g * _CH, _CH)])
            return 0

        lax.fori_loop(0, chunks_per_w, body, 0)

    return k


def kernel(tokens, table):
    b, l = tokens.shape
    v, d = table.shape
    n = b * l
    scaled = _scale_table(table, math.sqrt(d))
    idx = tokens.reshape(_NW, n // (_NW * _CH), _CH).astype(jnp.int32)
    out = _make_sc_gather(n, v, d)(scaled, idx)
    return out.reshape(b, l, d)


# speedup vs baseline: 7.2722x; 1.3564x over previous
import functools
import math

import jax
import jax.numpy as jnp
from jax import lax
from jax.experimental import pallas as pl
from jax.experimental.pallas import tpu as pltpu
from jax.experimental.pallas import tpu_sc as plsc


_NC = 2
_NS = 16
_NW = _NC * _NS

_CH = 128


def _scale_table(table, scale):
    v, d = table.shape
    block = 800
    assert v % block == 0

    def body(t_ref, o_ref):
        o_ref[...] = t_ref[...] * scale

    return pl.pallas_call(
        body,
        grid=(v // block,),
        in_specs=[pl.BlockSpec((block, d), lambda i: (i, 0))],
        out_specs=pl.BlockSpec((block, d), lambda i: (i, 0)),
        out_shape=jax.ShapeDtypeStruct((v, d), table.dtype),
    )(table)


_NB = 4


def _make_sc_gather(n, v, d):
    assert n % (_NW * _CH * _NB) == 0
    chunks_per_w = n // (_NW * _CH)
    mesh = plsc.VectorSubcoreMesh(core_axis_name="c", subcore_axis_name="s")

    @functools.partial(
        pl.kernel,
        mesh=mesh,
        out_type=jax.ShapeDtypeStruct((n, d), jnp.float32),
        scratch_types=[
            pltpu.VMEM((chunks_per_w, _CH), jnp.int32),
            [pltpu.VMEM((_CH, d), jnp.float32)] * _NB,
            [pltpu.SemaphoreType.DMA] * _NB,
            [pltpu.SemaphoreType.DMA] * _NB,
        ],
    )
    def k(table_hbm, idx_hbm, out_hbm, idx_v, rows, gsems, ssems):
        wid = lax.axis_index("s") * _NC + lax.axis_index("c")
        base = wid * (chunks_per_w * _CH)
        pltpu.sync_copy(idx_hbm.at[wid], idx_v)

        for b in range(_NB):
            pltpu.async_copy(table_hbm.at[idx_v.at[b]], rows[b], gsems[b])

        def body(g, _):
            for b in range(_NB):
                i = g + b
                pltpu.make_async_copy(
                    table_hbm.at[idx_v.at[0]], rows[b], gsems[b]
                ).wait()
                pltpu.async_copy(
                    rows[b], out_hbm.at[pl.ds(base + i * _CH, _CH)], ssems[b]
                )
            for b in range(_NB):
                i_next = g + _NB + b

                @pl.when(i_next < chunks_per_w)
                def _():
                    pltpu.make_async_copy(
                        rows[b], out_hbm.at[pl.ds(base, _CH)], ssems[b]
                    ).wait()
                    pltpu.async_copy(
                        table_hbm.at[idx_v.at[i_next]], rows[b], gsems[b]
                    )

            return 0

        lax.fori_loop(0, chunks_per_w // _NB, lambda g, c: body(g * _NB, c), 0)

        for b in range(_NB):
            pltpu.make_async_copy(
                rows[b], out_hbm.at[pl.ds(base, _CH)], ssems[b]
            ).wait()

    return k


def kernel(tokens, table):
    b, l = tokens.shape
    v, d = table.shape
    n = b * l
    scaled = _scale_table(table, math.sqrt(d))
    assert n % (_NW * _CH * _NB) == 0
    idx = tokens.reshape(_NW, n // (_NW * _CH), _CH).astype(jnp.int32)
    out = _make_sc_gather(n, v, d)(scaled, idx)
    return out.reshape(b, l, d)
